# baseline (speedup 1.0000x reference)
import functools

import jax
import jax.numpy as jnp
from jax import lax
from jax.experimental import pallas as pl
from jax.experimental.pallas import tpu as pltpu
from jax.experimental.pallas import tpu_sc as plsc

_NC = 2
_NS = 16
_NW = _NC * _NS
_CB = 2
_CHUNK = 128


@functools.partial(jax.jit, static_argnames=("n_chunks", "sp"))
def _sc_gather(table, idx_flat, *, n_chunks, sp):
    d = table.shape[1]
    cpw = n_chunks // _NW
    opc = _CB * sp
    mesh = plsc.VectorSubcoreMesh(core_axis_name="c", subcore_axis_name="s")

    @functools.partial(
        pl.kernel,
        mesh=mesh,
        out_type=jax.ShapeDtypeStruct((n_chunks * opc, d), jnp.float32),
        scratch_types=[
            pltpu.VMEM((cpw * _CHUNK,), jnp.int32),
            pltpu.VMEM((_CHUNK, d), jnp.float32),
            pltpu.SemaphoreType.DMA,
        ],
    )
    def k(table_hbm, idx_hbm, out_hbm, idx_v, rows_v, sem):
        wid = lax.axis_index("s") * _NC + lax.axis_index("c")
        pltpu.sync_copy(idx_hbm.at[pl.ds(wid * cpw * _CHUNK, cpw * _CHUNK)], idx_v)
        base = wid * cpw * opc

        @pl.loop(0, cpw)
        def _(i):
            pltpu.async_copy(
                table_hbm.at[idx_v.at[pl.ds(i * _CHUNK, _CHUNK)]], rows_v, sem
            ).wait()
            pltpu.sync_copy(
                rows_v.at[pl.ds(0, opc)], out_hbm.at[pl.ds(base + i * opc, opc)]
            )

    return k(table, idx_flat)


def kernel(inputs, kernel):
    table = kernel
    ids = inputs.astype(jnp.int32)
    n_rows, s = ids.shape
    d = table.shape[1]
    sp = -(-s // 8) * 8
    assert _CB * sp <= _CHUNK and n_rows % (_NW * _CB) == 0
    idsp = jnp.pad(ids, ((0, 0), (0, sp - s)))
    idsp = idsp.reshape(n_rows // _CB, _CB * sp)
    idsp = jnp.pad(idsp, ((0, 0), (0, _CHUNK - _CB * sp)))
    n_chunks = idsp.shape[0]
    out = _sc_gather(table, idsp.reshape(-1), n_chunks=n_chunks, sp=sp)
    return out.reshape(n_rows, sp, d)[:, :s, :]


# speedup vs baseline: 1.6268x; 1.6268x over previous
import functools

import jax
import jax.numpy as jnp
from jax import lax
from jax.experimental import pallas as pl
from jax.experimental.pallas import tpu as pltpu
from jax.experimental.pallas import tpu_sc as plsc

_NC = 2
_NS = 16
_NW = _NC * _NS
_CHUNK = 128


@functools.partial(jax.jit, static_argnames=("rows_per_w",))
def _sc_gather(table, idx_flat, *, rows_per_w):
    n_idx = idx_flat.shape[0]
    d = table.shape[1]
    n_chunks = rows_per_w // _CHUNK
    mesh = plsc.VectorSubcoreMesh(core_axis_name="c", subcore_axis_name="s")

    @functools.partial(
        pl.kernel,
        mesh=mesh,
        out_type=jax.ShapeDtypeStruct((n_idx, d), jnp.float32),
        scratch_types=[
            pltpu.VMEM((rows_per_w,), jnp.int32),
            pltpu.VMEM((_CHUNK, d), jnp.float32),
            pltpu.SemaphoreType.DMA,
        ],
    )
    def k(table_hbm, idx_hbm, out_hbm, idx_v, rows_v, sem):
        wid = lax.axis_index("s") * _NC + lax.axis_index("c")
        base = wid * rows_per_w
        pltpu.sync_copy(idx_hbm.at[pl.ds(base, rows_per_w)], idx_v)

        @pl.loop(0, n_chunks)
        def _(i):
            pltpu.async_copy(
                table_hbm.at[idx_v.at[pl.ds(i * _CHUNK, _CHUNK)]], rows_v, sem
            ).wait()
            pltpu.sync_copy(rows_v, out_hbm.at[pl.ds(base + i * _CHUNK, _CHUNK)])

    return k(table, idx_flat)


def kernel(inputs, kernel):
    table = kernel
    ids = inputs.astype(jnp.int32)
    n_rows, s = ids.shape
    d = table.shape[1]
    sp = -(-s // 8) * 8
    idsp = jnp.pad(ids, ((0, 0), (0, sp - s)))
    n = n_rows * sp
    assert n % (_NW * _CHUNK) == 0
    out = _sc_gather(table, idsp.reshape(-1), rows_per_w=n // _NW)
    return out.reshape(n_rows, sp, d)[:, :s, :]


# speedup vs baseline: 4.2557x; 2.6159x over previous
import functools

import jax
import jax.numpy as jnp
from jax import lax
from jax.experimental import pallas as pl
from jax.experimental.pallas import tpu as pltpu
from jax.experimental.pallas import tpu_sc as plsc

_NC = 2
_NS = 16
_NW = _NC * _NS
_CHUNK = 128


@functools.partial(jax.jit, static_argnames=("rows_per_w",))
def _sc_gather(table, idx_flat, *, rows_per_w):
    n_idx = idx_flat.shape[0]
    d = table.shape[1]
    n_chunks = rows_per_w // _CHUNK
    mesh = plsc.VectorSubcoreMesh(core_axis_name="c", subcore_axis_name="s")

    @functools.partial(
        pl.kernel,
        mesh=mesh,
        out_type=jax.ShapeDtypeStruct((n_idx, d), jnp.float32),
        scratch_types=[
            pltpu.VMEM((rows_per_w,), jnp.int32),
            pltpu.VMEM((_CHUNK, d), jnp.float32),
            pltpu.SemaphoreType.DMA,
        ],
    )
    def k(table_hbm, idx_hbm, out_hbm, idx_v, rows_v, sem):
        wid = lax.axis_index("s") * _NC + lax.axis_index("c")
        base = wid * rows_per_w
        pltpu.sync_copy(idx_hbm.at[pl.ds(base, rows_per_w)], idx_v)

        @pl.loop(0, n_chunks)
        def _(i):
            pltpu.async_copy(
                table_hbm.at[idx_v.at[pl.ds(i * _CHUNK, _CHUNK)]], rows_v, sem
            ).wait()
            pltpu.sync_copy(rows_v, out_hbm.at[pl.ds(base + i * _CHUNK, _CHUNK)])

    return k(table, idx_flat)


def kernel(inputs, kernel):
    table = kernel
    ids = inputs.astype(jnp.int32)
    n_rows, s = ids.shape
    d = table.shape[1]
    sp = -(-s // 8) * 8
    v = table.shape[0]
    pad_ids = (
        jnp.arange(n_rows, dtype=jnp.int32)[:, None] * (sp - s)
        + jnp.arange(sp - s, dtype=jnp.int32)[None, :]
    ) % v
    idsp = jnp.concatenate([ids, pad_ids], axis=1)
    n = n_rows * sp
    assert n % (_NW * _CHUNK) == 0
    out = _sc_gather(table, idsp.reshape(-1), rows_per_w=n // _NW)
    return out.reshape(n_rows, sp, d)[:, :s, :]
